# baseline (speedup 1.0000x reference)
import jax
import jax.numpy as jnp
from jax import lax
from jax.experimental import pallas as pl
from jax.experimental.pallas import tpu as pltpu
from jax.experimental.pallas import tpu_sc as plsc

_N = 10000
_E = 320000
_DIN = 128
_DH = 8
_DOUT = 64
_W = 16
_NP = _N + 8

_NC = 2
_NS = 16
_NW = _NC * _NS
_B = 128
_K = 80
_EP = _NW * _K * _B
_K0 = 80
_K1 = 80
_E0 = _NS * _K0 * _B
_AT = 10
_AR = _N // _AT

_f32 = jnp.float32


_R = 4
_R0 = 2
_R1 = 4


def _sc_loop(feat_hbm, acc_sh, si_v, di_v, g_vs, sg, myk, ring):
    for b in range(ring):
        pltpu.async_copy(feat_hbm.at[si_v.at[b, 0]], g_vs[b], sg[b])

    def _group(g, carry):
        for b in range(ring):
            j = ring * g + b
            pltpu.make_async_copy(feat_hbm.at[si_v.at[0, 0]], g_vs[b],
                                  sg[b]).wait()
            pltpu.sync_copy(g_vs[b], acc_sh.at[di_v.at[j, 0]], add=True)

            @pl.when(j + ring < myk)
            def _prefetch(j=j, b=b):
                pltpu.async_copy(feat_hbm.at[si_v.at[j + ring, 0]],
                                 g_vs[b], sg[b])
        return carry
    lax.fori_loop(0, myk // ring, _group, 0)


def _sc_body(src_hbm, dst_hbm, feat_hbm, out_hbm,
             si_v, di_v, g0_v, g1_v, g2_v, g3_v,
             stg_v, acc_sh, sg0, sg1, sg2, sg3):
    g_vs = (g0_v, g1_v, g2_v, g3_v)
    sg = (sg0, sg1, sg2, sg3)
    c = lax.axis_index("c")
    s = lax.axis_index("s")
    w = c * _NS + s

    pltpu.sync_copy(src_hbm.at[w], si_v)
    pltpu.sync_copy(dst_hbm.at[w], di_v)

    @pl.when(s < _AT)
    def _zero():
        def _z(i, carry):
            stg_v[i] = jnp.zeros((_W,), _f32)
            return carry
        lax.fori_loop(0, _AR, _z, 0, unroll=8)
        pltpu.sync_copy(stg_v, acc_sh.at[pl.ds(s * _AR, _AR)])
    plsc.subcore_barrier()

    @pl.when(c == 0)
    def _core0():
        _sc_loop(feat_hbm, acc_sh, si_v, di_v, g_vs, sg, _K0, _R0)

    @pl.when(c == 1)
    def _core1():
        _sc_loop(feat_hbm, acc_sh, si_v, di_v, g_vs, sg, _K1, _R1)
    plsc.subcore_barrier()

    @pl.when(s < _AT)
    def _pub():
        pltpu.sync_copy(acc_sh.at[pl.ds(s * _AR, _AR)],
                        out_hbm.at[c, pl.ds(s * _AR, _AR)])


_sc_aggregate_cache = []


def _sc_aggregate(src, dst, feat):
    if not _sc_aggregate_cache:
        _sc_aggregate_cache.append(pl.kernel(
            _sc_body,
            out_type=jax.ShapeDtypeStruct((_NC, _N, _W), _f32),
            mesh=plsc.VectorSubcoreMesh(core_axis_name="c", subcore_axis_name="s",
                                        num_cores=_NC, num_subcores=_NS),
            compiler_params=pltpu.CompilerParams(use_tc_tiling_on_sc=False),
            scratch_types=[
                pltpu.VMEM((_K0, 1, _B), jnp.int32),
                pltpu.VMEM((_K0, 1, _B), jnp.int32),
            ] + [pltpu.VMEM((_B, _W), _f32)] * _R
              + [pltpu.VMEM((_AR, _W), _f32),
                 pltpu.VMEM_SHARED((_NP, _W), _f32)]
              + [pltpu.SemaphoreType.DMA] * _R,
        ))
    return _sc_aggregate_cache[0](src, dst, feat)


def _tc1_body(x_ref, wct_ref, y_ref, xr_ref):
    y = jnp.dot(x_ref[...], wct_ref[...], preferred_element_type=_f32)
    ye = jnp.concatenate(
        [y[:, :_DH], jnp.ones((_N, 1), _f32), jnp.zeros((_N, _W - _DH - 1), _f32)],
        axis=1)
    y_ref[...] = jnp.concatenate([ye, jnp.zeros((_NP - _N, _W), _f32)], axis=0)
    xr_ref[...] = y[:, _DH:]


_tc1 = pl.pallas_call(
    _tc1_body,
    out_shape=(jax.ShapeDtypeStruct((_NP, _W), _f32),
               jax.ShapeDtypeStruct((_N, _DH), _f32)),
)


def _tc2_body(p_ref, xr_ref, b1_ref, h_ref):
    acc = p_ref[0] + p_ref[1]
    deg = jnp.maximum(acc[:, _DH:_DH + 1], 1.0)
    t = jnp.maximum(acc[:, :_DH] / deg + b1_ref[...] + xr_ref[...], 0.0)
    he = jnp.concatenate(
        [t, jnp.ones((_N, 1), _f32), jnp.zeros((_N, _W - _DH - 1), _f32)],
        axis=1)
    h_ref[...] = jnp.concatenate([he, jnp.zeros((_NP - _N, _W), _f32)], axis=0)


_tc2 = pl.pallas_call(
    _tc2_body,
    out_shape=jax.ShapeDtypeStruct((_NP, _W), _f32),
)


def _tc3_body(p_ref, h_ref, w2l_ref, w2r_ref, b2_ref, o_ref):
    acc = p_ref[0] + p_ref[1]
    deg = jnp.maximum(acc[:, _DH:_DH + 1], 1.0)
    col = lax.broadcasted_iota(jnp.int32, (_N, _W), 1)
    z = jnp.where(col < _DH, acc / deg, 0.0)
    o = (jnp.dot(z, w2l_ref[...], preferred_element_type=_f32)
         + jnp.dot(h_ref[:_N, :], w2r_ref[...], preferred_element_type=_f32)
         + b2_ref[...])
    m = jnp.max(o, axis=1, keepdims=True)
    o_ref[...] = o - m - jnp.log(jnp.sum(jnp.exp(o - m), axis=1, keepdims=True))


_tc3 = pl.pallas_call(
    _tc3_body,
    out_shape=jax.ShapeDtypeStruct((_N, _DOUT), _f32),
)


def kernel(x, edge_index, W1l, b1l, W1r, W2l, b2l, W2r):
    pad = jnp.full((2, _EP - _E), _N, dtype=jnp.int32)
    eip = jnp.concatenate([edge_index, pad], axis=1)
    e0 = eip[:, :_E0].reshape(2, _NS, _K0, 1, _B)
    e1 = eip[:, _E0:].reshape(2, _NS, _K1, 1, _B)
    e1 = jnp.pad(e1, ((0, 0), (0, 0), (0, _K0 - _K1), (0, 0), (0, 0)),
                 constant_values=_N)
    ei4 = jnp.concatenate([e0, e1], axis=1)
    src = ei4[0]
    dst = ei4[1]
    wct = jnp.concatenate([W1l, W1r], axis=0).T
    b1e = b1l.reshape(1, _DH)
    w2lt = jnp.pad(W2l.T, ((0, _W - _DH), (0, 0)))
    w2rt = jnp.pad(W2r.T, ((0, _W - _DH), (0, 0)))
    b2e = b2l.reshape(1, _DOUT)

    y_ext, xr = _tc1(x, wct)
    p1 = _sc_aggregate(src, dst, y_ext)
    h_ext = _tc2(p1, xr, b1e)
    p2 = _sc_aggregate(src, dst, h_ext)
    return _tc3(p2, h_ext, w2lt, w2rt, b2e)


# speedup vs baseline: 1.0081x; 1.0081x over previous
import jax
import jax.numpy as jnp
from jax import lax
from jax.experimental import pallas as pl
from jax.experimental.pallas import tpu as pltpu
from jax.experimental.pallas import tpu_sc as plsc

_N = 10000
_E = 320000
_DIN = 128
_DH = 8
_DOUT = 64
_W = 16
_NP = _N + 8

_NC = 2
_NS = 16
_NW = _NC * _NS
_B = 128
_K = 80
_EP = _NW * _K * _B
_K0 = 80
_K1 = 80
_E0 = _NS * _K0 * _B
_AT = 10
_AR = _N // _AT

_f32 = jnp.float32


_R = 4


def _sc_body(src_hbm, dst_hbm, feat_hbm, out_hbm,
             si_v, di_v, g0_v, g1_v, g2_v, g3_v,
             stg_v, acc_sh, sg0, sg1, sg2, sg3):
    g_vs = (g0_v, g1_v, g2_v, g3_v)
    sg = (sg0, sg1, sg2, sg3)
    c = lax.axis_index("c")
    s = lax.axis_index("s")
    w = c * _NS + s

    pltpu.sync_copy(src_hbm.at[w], si_v)
    pltpu.sync_copy(dst_hbm.at[w], di_v)
    for b in range(_R):
        pltpu.async_copy(feat_hbm.at[si_v.at[b, 0]], g_vs[b], sg[b])

    @pl.when(s < _AT)
    def _zero():
        def _z(i, carry):
            stg_v[i] = jnp.zeros((_W,), _f32)
            return carry
        lax.fori_loop(0, _AR, _z, 0, unroll=8)
        pltpu.sync_copy(stg_v, acc_sh.at[pl.ds(s * _AR, _AR)])
    plsc.subcore_barrier()

    def _group(g, carry):
        for b in range(_R):
            j = _R * g + b
            pltpu.make_async_copy(feat_hbm.at[si_v.at[0, 0]], g_vs[b],
                                  sg[b]).wait()
            pltpu.sync_copy(g_vs[b], acc_sh.at[di_v.at[j, 0]], add=True)

            @pl.when(j + _R < _K)
            def _prefetch(j=j, b=b):
                pltpu.async_copy(feat_hbm.at[si_v.at[j + _R, 0]],
                                 g_vs[b], sg[b])
        return carry
    lax.fori_loop(0, _K // _R, _group, 0)
    plsc.subcore_barrier()

    @pl.when(s < _AT)
    def _pub():
        pltpu.sync_copy(acc_sh.at[pl.ds(s * _AR, _AR)],
                        out_hbm.at[c, pl.ds(s * _AR, _AR)])


_sc_aggregate_cache = []


def _sc_aggregate(src, dst, feat):
    if not _sc_aggregate_cache:
        _sc_aggregate_cache.append(pl.kernel(
            _sc_body,
            out_type=jax.ShapeDtypeStruct((_NC, _N, _W), _f32),
            mesh=plsc.VectorSubcoreMesh(core_axis_name="c", subcore_axis_name="s",
                                        num_cores=_NC, num_subcores=_NS),
            compiler_params=pltpu.CompilerParams(use_tc_tiling_on_sc=False),
            scratch_types=[
                pltpu.VMEM((_K0, 1, _B), jnp.int32),
                pltpu.VMEM((_K0, 1, _B), jnp.int32),
            ] + [pltpu.VMEM((_B, _W), _f32)] * _R
              + [pltpu.VMEM((_AR, _W), _f32),
                 pltpu.VMEM_SHARED((_NP, _W), _f32)]
              + [pltpu.SemaphoreType.DMA] * _R,
        ))
    return _sc_aggregate_cache[0](src, dst, feat)


def _tc1_body(x_ref, wct_ref, y_ref, xr_ref):
    y = jnp.dot(x_ref[...], wct_ref[...], preferred_element_type=_f32)
    ye = jnp.concatenate(
        [y[:, :_DH], jnp.ones((_N, 1), _f32), jnp.zeros((_N, _W - _DH - 1), _f32)],
        axis=1)
    y_ref[...] = jnp.concatenate([ye, jnp.zeros((_NP - _N, _W), _f32)], axis=0)
    xr_ref[...] = y[:, _DH:]


_tc1 = pl.pallas_call(
    _tc1_body,
    out_shape=(jax.ShapeDtypeStruct((_NP, _W), _f32),
               jax.ShapeDtypeStruct((_N, _DH), _f32)),
)


def _tc2_body(p_ref, xr_ref, b1_ref, h_ref):
    acc = p_ref[0] + p_ref[1]
    deg = jnp.maximum(acc[:, _DH:_DH + 1], 1.0)
    t = jnp.maximum(acc[:, :_DH] / deg + b1_ref[...] + xr_ref[...], 0.0)
    he = jnp.concatenate(
        [t, jnp.ones((_N, 1), _f32), jnp.zeros((_N, _W - _DH - 1), _f32)],
        axis=1)
    h_ref[...] = jnp.concatenate([he, jnp.zeros((_NP - _N, _W), _f32)], axis=0)


_tc2 = pl.pallas_call(
    _tc2_body,
    out_shape=jax.ShapeDtypeStruct((_NP, _W), _f32),
)


def _tc3_body(p_ref, h_ref, w2l_ref, w2r_ref, b2_ref, o_ref):
    acc = p_ref[0] + p_ref[1]
    deg = jnp.maximum(acc[:, _DH:_DH + 1], 1.0)
    col = lax.broadcasted_iota(jnp.int32, (_N, _W), 1)
    z = jnp.where(col < _DH, acc / deg, 0.0)
    o = (jnp.dot(z, w2l_ref[...], preferred_element_type=_f32)
         + jnp.dot(h_ref[:_N, :], w2r_ref[...], preferred_element_type=_f32)
         + b2_ref[...])
    m = jnp.max(o, axis=1, keepdims=True)
    o_ref[...] = o - m - jnp.log(jnp.sum(jnp.exp(o - m), axis=1, keepdims=True))


_tc3 = pl.pallas_call(
    _tc3_body,
    out_shape=jax.ShapeDtypeStruct((_N, _DOUT), _f32),
)


def kernel(x, edge_index, W1l, b1l, W1r, W2l, b2l, W2r):
    pad = jnp.full((2, _EP - _E), _N, dtype=jnp.int32)
    eip = jnp.concatenate([edge_index, pad], axis=1)
    e0 = eip[:, :_E0].reshape(2, _NS, _K0, 1, _B)
    e1 = eip[:, _E0:].reshape(2, _NS, _K1, 1, _B)
    e1 = jnp.pad(e1, ((0, 0), (0, 0), (0, _K0 - _K1), (0, 0), (0, 0)),
                 constant_values=_N)
    ei4 = jnp.concatenate([e0, e1], axis=1)
    src = ei4[0]
    dst = ei4[1]
    wct = jnp.concatenate([W1l, W1r], axis=0).T
    b1e = b1l.reshape(1, _DH)
    w2lt = jnp.pad(W2l.T, ((0, _W - _DH), (0, 0)))
    w2rt = jnp.pad(W2r.T, ((0, _W - _DH), (0, 0)))
    b2e = b2l.reshape(1, _DOUT)

    y_ext, xr = _tc1(x, wct)
    p1 = _sc_aggregate(src, dst, y_ext)
    h_ext = _tc2(p1, xr, b1e)
    p2 = _sc_aggregate(src, dst, h_ext)
    return _tc3(p2, h_ext, w2lt, w2rt, b2e)


# speedup vs baseline: 1.7253x; 1.7114x over previous
import jax
import jax.numpy as jnp
from jax import lax
from jax.experimental import pallas as pl
from jax.experimental.pallas import tpu as pltpu
from jax.experimental.pallas import tpu_sc as plsc

_N = 10000
_E = 320000
_DIN = 128
_DH = 8
_DOUT = 64
_W = 16
_NP = _N + 8

_NC = 2
_NS = 16
_NW = _NC * _NS
_B = 128
_K = 80
_EP = _NW * _K * _B
_K0 = 80
_K1 = 80
_E0 = _NS * _K0 * _B
_AT = 10
_AR = _N // _AT

_f32 = jnp.float32


_R = 4


def _sc_body(src_hbm, dst_hbm, feat_hbm, out_hbm,
             si_v, di_v, g0_v, g1_v, g2_v, g3_v,
             stg_v, acc_sh, tab_sh, sg0, sg1, sg2, sg3):
    g_vs = (g0_v, g1_v, g2_v, g3_v)
    sg = (sg0, sg1, sg2, sg3)
    c = lax.axis_index("c")
    s = lax.axis_index("s")
    w = c * _NS + s

    pltpu.sync_copy(src_hbm.at[w], si_v)
    pltpu.sync_copy(dst_hbm.at[w], di_v)

    @pl.when(s < _AT)
    def _zero():
        pltpu.sync_copy(feat_hbm.at[pl.ds(s * _AR, _AR)],
                        tab_sh.at[pl.ds(s * _AR, _AR)])

        def _z(i, carry):
            stg_v[i] = jnp.zeros((_W,), _f32)
            return carry
        lax.fori_loop(0, _AR, _z, 0, unroll=8)
        pltpu.sync_copy(stg_v, acc_sh.at[pl.ds(s * _AR, _AR)])
    plsc.subcore_barrier()

    for b in range(_R):
        pltpu.async_copy(tab_sh.at[si_v.at[b, 0]], g_vs[b], sg[b])

    def _group(g, carry):
        for b in range(_R):
            j = _R * g + b
            pltpu.make_async_copy(tab_sh.at[si_v.at[0, 0]], g_vs[b],
                                  sg[b]).wait()
            pltpu.sync_copy(g_vs[b], acc_sh.at[di_v.at[j, 0]], add=True)

            @pl.when(j + _R < _K)
            def _prefetch(j=j, b=b):
                pltpu.async_copy(tab_sh.at[si_v.at[j + _R, 0]],
                                 g_vs[b], sg[b])
        return carry
    lax.fori_loop(0, _K // _R, _group, 0)
    plsc.subcore_barrier()

    @pl.when(s < _AT)
    def _pub():
        pltpu.sync_copy(acc_sh.at[pl.ds(s * _AR, _AR)],
                        out_hbm.at[c, pl.ds(s * _AR, _AR)])


_sc_aggregate_cache = []


def _sc_aggregate(src, dst, feat):
    if not _sc_aggregate_cache:
        _sc_aggregate_cache.append(pl.kernel(
            _sc_body,
            out_type=jax.ShapeDtypeStruct((_NC, _N, _W), _f32),
            mesh=plsc.VectorSubcoreMesh(core_axis_name="c", subcore_axis_name="s",
                                        num_cores=_NC, num_subcores=_NS),
            compiler_params=pltpu.CompilerParams(use_tc_tiling_on_sc=False),
            scratch_types=[
                pltpu.VMEM((_K0, 1, _B), jnp.int32),
                pltpu.VMEM((_K0, 1, _B), jnp.int32),
            ] + [pltpu.VMEM((_B, _W), _f32)] * _R
              + [pltpu.VMEM((_AR, _W), _f32),
                 pltpu.VMEM_SHARED((_NP, _W), _f32),
                 pltpu.VMEM_SHARED((_NP, _W), _f32)]
              + [pltpu.SemaphoreType.DMA] * _R,
        ))
    return _sc_aggregate_cache[0](src, dst, feat)


def _tc1_body(x_ref, wct_ref, y_ref, xr_ref):
    y = jnp.dot(x_ref[...], wct_ref[...], preferred_element_type=_f32)
    ye = jnp.concatenate(
        [y[:, :_DH], jnp.ones((_N, 1), _f32), jnp.zeros((_N, _W - _DH - 1), _f32)],
        axis=1)
    y_ref[...] = jnp.concatenate([ye, jnp.zeros((_NP - _N, _W), _f32)], axis=0)
    xr_ref[...] = y[:, _DH:]


_tc1 = pl.pallas_call(
    _tc1_body,
    out_shape=(jax.ShapeDtypeStruct((_NP, _W), _f32),
               jax.ShapeDtypeStruct((_N, _DH), _f32)),
)


def _tc2_body(p_ref, xr_ref, b1_ref, h_ref):
    acc = p_ref[0] + p_ref[1]
    deg = jnp.maximum(acc[:, _DH:_DH + 1], 1.0)
    t = jnp.maximum(acc[:, :_DH] / deg + b1_ref[...] + xr_ref[...], 0.0)
    he = jnp.concatenate(
        [t, jnp.ones((_N, 1), _f32), jnp.zeros((_N, _W - _DH - 1), _f32)],
        axis=1)
    h_ref[...] = jnp.concatenate([he, jnp.zeros((_NP - _N, _W), _f32)], axis=0)


_tc2 = pl.pallas_call(
    _tc2_body,
    out_shape=jax.ShapeDtypeStruct((_NP, _W), _f32),
)


def _tc3_body(p_ref, h_ref, w2l_ref, w2r_ref, b2_ref, o_ref):
    acc = p_ref[0] + p_ref[1]
    deg = jnp.maximum(acc[:, _DH:_DH + 1], 1.0)
    col = lax.broadcasted_iota(jnp.int32, (_N, _W), 1)
    z = jnp.where(col < _DH, acc / deg, 0.0)
    o = (jnp.dot(z, w2l_ref[...], preferred_element_type=_f32)
         + jnp.dot(h_ref[:_N, :], w2r_ref[...], preferred_element_type=_f32)
         + b2_ref[...])
    m = jnp.max(o, axis=1, keepdims=True)
    o_ref[...] = o - m - jnp.log(jnp.sum(jnp.exp(o - m), axis=1, keepdims=True))


_tc3 = pl.pallas_call(
    _tc3_body,
    out_shape=jax.ShapeDtypeStruct((_N, _DOUT), _f32),
)


def kernel(x, edge_index, W1l, b1l, W1r, W2l, b2l, W2r):
    pad = jnp.full((2, _EP - _E), _N, dtype=jnp.int32)
    eip = jnp.concatenate([edge_index, pad], axis=1)
    e0 = eip[:, :_E0].reshape(2, _NS, _K0, 1, _B)
    e1 = eip[:, _E0:].reshape(2, _NS, _K1, 1, _B)
    e1 = jnp.pad(e1, ((0, 0), (0, 0), (0, _K0 - _K1), (0, 0), (0, 0)),
                 constant_values=_N)
    ei4 = jnp.concatenate([e0, e1], axis=1)
    src = ei4[0]
    dst = ei4[1]
    wct = jnp.concatenate([W1l, W1r], axis=0).T
    b1e = b1l.reshape(1, _DH)
    w2lt = jnp.pad(W2l.T, ((0, _W - _DH), (0, 0)))
    w2rt = jnp.pad(W2r.T, ((0, _W - _DH), (0, 0)))
    b2e = b2l.reshape(1, _DOUT)

    y_ext, xr = _tc1(x, wct)
    p1 = _sc_aggregate(src, dst, y_ext)
    h_ext = _tc2(p1, xr, b1e)
    p2 = _sc_aggregate(src, dst, h_ext)
    return _tc3(p2, h_ext, w2lt, w2rt, b2e)
